# baseline (speedup 1.0000x reference)
import functools

import jax
import jax.numpy as jnp
from jax import lax
from jax.experimental import pallas as pl
from jax.experimental.pallas import tpu as pltpu
from jax.experimental.pallas import tpu_sc as plsc

NC, NS = 2, 16
NW = NC * NS
B = 16384 * 50
D = 64
V = 1_000_000
CHUNK = 128
PER_W = B // NW
NBLK = PER_W // CHUNK

VBLK = V // CHUNK
VTAIL = V - VBLK * CHUNK
SLOTS0 = 246

_mesh = plsc.VectorSubcoreMesh(
    core_axis_name="c", subcore_axis_name="s", num_cores=NC, num_subcores=NS
)


def _iota16():
    return lax.iota(jnp.int32, 16)


@functools.partial(
    pl.kernel,
    out_type=jax.ShapeDtypeStruct((V // 2, 2 * D), jnp.float32),
    mesh=_mesh,
    scratch_types=[
        pltpu.VMEM((D, CHUNK), jnp.float32),
        pltpu.VMEM((D, CHUNK), jnp.float32),
        pltpu.VMEM((D, CHUNK), jnp.float32),
        pltpu.VMEM((D, CHUNK), jnp.float32),
        pltpu.VMEM((D, D), jnp.float32),
        pltpu.VMEM((D // 2, CHUNK), jnp.float32),
        pltpu.SemaphoreType.DMA,
        pltpu.SemaphoreType.DMA,
        pltpu.SemaphoreType.DMA,
        pltpu.SemaphoreType.DMA,
    ],
    compiler_params=pltpu.CompilerParams(use_tc_tiling_on_sc=True, needs_layout_passes=False),
)
def _table_transpose(tt_hbm, t2_hbm, stage0, stage1, tbuf0, tbuf1,
                     stail, ttail, rsem0, rsem1, wsem0, wsem1):
    wid = lax.axis_index("s") * NC + lax.axis_index("c")
    nb_w = jnp.where(wid < VBLK - (VBLK // NW) * NW, VBLK // NW + 1, VBLK // NW)
    iota = _iota16()
    stages = (stage0, stage1)
    tbufs = (tbuf0, tbuf1)
    rsems = (rsem0, rsem1)
    wsems = (wsem0, wsem1)

    def blk(t):
        return wid + NW * t

    def fire_read(t, stage, rsem):
        pltpu.async_copy(tt_hbm.at[:, pl.ds(blk(t) * CHUNK, CHUNK)], stage,
                         rsem)

    def transpose_pairs(src, dst, nq):
        for q in range(nq):
            for l0 in range(0, 2 * D, 16):
                h, c0 = divmod(l0, D)
                v = plsc.load_gather(
                    src, [iota + c0, jnp.full((16,), 2 * q + h, jnp.int32)])
                dst[q, pl.ds(l0, 16)] = v

    fire_read(0, stage0, rsem0)

    @pl.loop(0, SLOTS0, step=2)
    def _(t0):
        for p in range(2):
            t = t0 + p
            stage, tbuf, rsem, wsem = stages[p], tbufs[p], rsems[p], wsems[p]

            @pl.when(t < nb_w)
            def _():
                pltpu.make_async_copy(
                    tt_hbm.at[:, pl.ds(0, CHUNK)], stage, rsem
                ).wait()

                @pl.when(t + 1 < nb_w)
                def _():
                    fire_read(t + 1, stages[1 - p], rsems[1 - p])

                @pl.when(t >= 2)
                def _():
                    pltpu.make_async_copy(
                        tbuf, t2_hbm.at[pl.ds(0, D), :], wsem
                    ).wait()

                transpose_pairs(stage, tbuf, D)
                pltpu.async_copy(
                    tbuf, t2_hbm.at[pl.ds(blk(t) * (CHUNK // 2), D), :], wsem)

    for p in range(2):
        @pl.when(nb_w >= 2 - p)
        def _():
            pltpu.make_async_copy(
                tbufs[p], t2_hbm.at[pl.ds(0, D), :], wsems[p]
            ).wait()

    @pl.when(wid == 0)
    def _():
        pltpu.sync_copy(tt_hbm.at[:, pl.ds(VBLK * CHUNK, VTAIL)], stail)
        for q in range(VTAIL // 2):
            for l0 in range(0, 2 * D, 16):
                h, c0 = divmod(l0, D)
                v = plsc.load_gather(
                    stail, [_iota16() + c0,
                            jnp.full((16,), 2 * q + h, jnp.int32)])
                ttail[q, pl.ds(l0, 16)] = v
        pltpu.sync_copy(ttail, t2_hbm.at[pl.ds(VBLK * (CHUNK // 2),
                                               VTAIL // 2), :])


@functools.partial(
    pl.kernel,
    out_type=jax.ShapeDtypeStruct((50, 8, 128, 8, 128), jnp.float32),
    mesh=_mesh,
    scratch_types=[
        pltpu.VMEM((PER_W,), jnp.int32),
        pltpu.VMEM((CHUNK, D), jnp.float32),
        pltpu.VMEM((CHUNK, D), jnp.float32),
        pltpu.VMEM((8, 8, CHUNK), jnp.float32),
        pltpu.VMEM((8, 8, CHUNK), jnp.float32),
        pltpu.SemaphoreType.DMA,
        pltpu.SemaphoreType.DMA,
        pltpu.SemaphoreType.DMA,
        pltpu.SemaphoreType.DMA,
    ],
    compiler_params=pltpu.CompilerParams(use_tc_tiling_on_sc=False, needs_layout_passes=False),
)
def _gather(x_hbm, t_hbm, out_hbm, idx_v, rows0, rows1, tbuf0, tbuf1,
            gsem0, gsem1, wsem0, wsem1):
    wid = lax.axis_index("s") * NC + lax.axis_index("c")
    pltpu.sync_copy(x_hbm.at[pl.ds(wid * PER_W, PER_W)], idx_v)
    iota = _iota16()
    rows = (rows0, rows1)
    tbufs = (tbuf0, tbuf1)
    gsems = (gsem0, gsem1)
    wsems = (wsem0, wsem1)

    def fire_gather(t, dst, gsem):
        pltpu.async_copy(
            t_hbm.at[idx_v.at[pl.ds(t * CHUNK, CHUNK)]], dst, gsem)

    fire_gather(0, rows0, gsem0)

    @pl.loop(0, NBLK, step=2)
    def _(t0):
        for p in range(2):
            t = t0 + p
            row, tbuf, gsem, wsem = rows[p], tbufs[p], gsems[p], wsems[p]
            pltpu.make_async_copy(
                t_hbm.at[pl.ds(0, CHUNK)], row, gsem
            ).wait()

            @pl.when(t + 1 < NBLK)
            def _():
                fire_gather(t + 1, rows[1 - p], gsems[1 - p])

            @pl.when(t >= 2)
            def _():
                pltpu.make_async_copy(
                    tbuf, out_hbm.at[0, :, 0], wsem
                ).wait()

            for c in range(D):
                colv = jnp.full((16,), c, jnp.int32)
                for g in range(CHUNK // 16):
                    v = plsc.load_gather(row, [iota + g * 16, colv])
                    tbuf[c // 8, c % 8, pl.ds(g * 16, 16)] = v

            bl = wid * NBLK + t
            pltpu.async_copy(tbuf, out_hbm.at[bl // 128, :, bl % 128], wsem)

    for p in range(2):
        pltpu.make_async_copy(
            tbufs[p], out_hbm.at[0, :, 0], wsems[p]
        ).wait()


def kernel(x, table):
    t2 = _table_transpose(table.T)
    tl = t2.reshape(V, D)
    xtf = x.T.reshape(B).astype(jnp.int32)
    o5 = _gather(xtf, tl)
    return o5.transpose(2, 4, 0, 1, 3).reshape(x.shape[0], x.shape[1], D)


# speedup vs baseline: 1.5224x; 1.5224x over previous
import functools

import jax
import jax.numpy as jnp
from jax import lax
from jax.experimental import pallas as pl
from jax.experimental.pallas import tpu as pltpu
from jax.experimental.pallas import tpu_sc as plsc

NC, NS = 2, 16
NW = NC * NS
B = 16384 * 50
D = 64
V = 1_000_000
CHUNK = 128
PER_W = B // NW
NBLK = PER_W // CHUNK

VBLK = V // CHUNK
VTAIL = V - VBLK * CHUNK
SLOTS0 = 246

_mesh = plsc.VectorSubcoreMesh(
    core_axis_name="c", subcore_axis_name="s", num_cores=NC, num_subcores=NS
)


def _iota16():
    return lax.iota(jnp.int32, 16)


def _pair_transpose(src, dst, nil):
    iota = _iota16()
    for il0 in range(0, nil, 16):
        d0v = (iota + il0) // 2
        d1b = ((iota + il0) % 2) * D
        for c0 in range(0, D, 16):
            vs = [(src[c, pl.ds(il0, 16)], c) for c in range(c0, c0 + 16)]
            for v, c in vs:
                plsc.store_scatter(dst, [d0v, d1b + c], v)


def _block_transpose(row, tbuf):
    iota = _iota16()
    d0 = []
    d1 = []
    for k in range(4):
        d0.append((iota + k * 16) // 8)
        d1.append(((iota + k * 16) % 8) * CHUNK)
    for i0 in range(0, CHUNK, 4):
        vs = []
        for i in range(i0, i0 + 4):
            for k in range(4):
                vs.append((row[i, pl.ds(k * 16, 16)], k, i))
        for v, k, i in vs:
            plsc.store_scatter(tbuf, [d0[k], d1[k] + i], v)


@functools.partial(
    pl.kernel,
    out_type=jax.ShapeDtypeStruct((V // 2, 2 * D), jnp.float32),
    mesh=_mesh,
    scratch_types=[
        pltpu.VMEM((D, CHUNK), jnp.float32),
        pltpu.VMEM((D, CHUNK), jnp.float32),
        pltpu.VMEM((D, CHUNK), jnp.float32),
        pltpu.VMEM((D, CHUNK), jnp.float32),
        pltpu.VMEM((D, D), jnp.float32),
        pltpu.VMEM((D // 2, CHUNK), jnp.float32),
        pltpu.SemaphoreType.DMA,
        pltpu.SemaphoreType.DMA,
        pltpu.SemaphoreType.DMA,
        pltpu.SemaphoreType.DMA,
    ],
    compiler_params=pltpu.CompilerParams(use_tc_tiling_on_sc=True,
                                         needs_layout_passes=False),
)
def _table_transpose(tt_hbm, t2_hbm, stage0, stage1, tbuf0, tbuf1,
                     stail, ttail, rsem0, rsem1, wsem0, wsem1):
    wid = lax.axis_index("s") * NC + lax.axis_index("c")
    nb_w = jnp.where(wid < VBLK - (VBLK // NW) * NW, VBLK // NW + 1,
                     VBLK // NW)
    stages = (stage0, stage1)
    tbufs = (tbuf0, tbuf1)
    rsems = (rsem0, rsem1)
    wsems = (wsem0, wsem1)

    def blk(t):
        return wid + NW * t

    def fire_read(t, stage, rsem):
        pltpu.async_copy(tt_hbm.at[:, pl.ds(blk(t) * CHUNK, CHUNK)], stage,
                         rsem)

    fire_read(0, stage0, rsem0)

    @pl.loop(0, SLOTS0, step=2)
    def _(t0):
        for p in range(2):
            t = t0 + p
            stage, tbuf, rsem, wsem = stages[p], tbufs[p], rsems[p], wsems[p]

            @pl.when(t < nb_w)
            def _():
                pltpu.make_async_copy(
                    tt_hbm.at[:, pl.ds(0, CHUNK)], stage, rsem
                ).wait()

                @pl.when(t + 1 < nb_w)
                def _():
                    fire_read(t + 1, stages[1 - p], rsems[1 - p])

                @pl.when(t >= 2)
                def _():
                    pltpu.make_async_copy(
                        tbuf, t2_hbm.at[pl.ds(0, D), :], wsem
                    ).wait()

                _pair_transpose(stage, tbuf, CHUNK)
                pltpu.async_copy(
                    tbuf, t2_hbm.at[pl.ds(blk(t) * (CHUNK // 2), D), :], wsem)

    for p in range(2):
        @pl.when(nb_w >= 2 - p)
        def _():
            pltpu.make_async_copy(
                tbufs[p], t2_hbm.at[pl.ds(0, D), :], wsems[p]
            ).wait()

    @pl.when(wid == 0)
    def _():
        pltpu.sync_copy(tt_hbm.at[:, pl.ds(VBLK * CHUNK, VTAIL)], stail)
        _pair_transpose(stail, ttail, VTAIL)
        pltpu.sync_copy(ttail, t2_hbm.at[pl.ds(VBLK * (CHUNK // 2),
                                               VTAIL // 2), :])


@functools.partial(
    pl.kernel,
    out_type=jax.ShapeDtypeStruct((50, 8, 128, 1024), jnp.float32),
    mesh=_mesh,
    scratch_types=[
        pltpu.VMEM((PER_W,), jnp.int32),
        pltpu.VMEM((CHUNK, D), jnp.float32),
        pltpu.VMEM((CHUNK, D), jnp.float32),
        pltpu.VMEM((8, 1024), jnp.float32),
        pltpu.VMEM((8, 1024), jnp.float32),
        pltpu.SemaphoreType.DMA,
        pltpu.SemaphoreType.DMA,
        pltpu.SemaphoreType.DMA,
        pltpu.SemaphoreType.DMA,
    ],
    compiler_params=pltpu.CompilerParams(use_tc_tiling_on_sc=False,
                                         needs_layout_passes=False),
)
def _gather(x_hbm, t_hbm, out_hbm, idx_v, rows0, rows1, tbuf0, tbuf1,
            gsem0, gsem1, wsem0, wsem1):
    wid = lax.axis_index("s") * NC + lax.axis_index("c")
    pltpu.sync_copy(x_hbm.at[pl.ds(wid * PER_W, PER_W)], idx_v)
    rows = (rows0, rows1)
    tbufs = (tbuf0, tbuf1)
    gsems = (gsem0, gsem1)
    wsems = (wsem0, wsem1)

    def fire_gather(t, dst, gsem):
        pltpu.async_copy(
            t_hbm.at[idx_v.at[pl.ds(t * CHUNK, CHUNK)]], dst, gsem)

    fire_gather(0, rows0, gsem0)

    @pl.loop(0, NBLK, step=2)
    def _(t0):
        for p in range(2):
            t = t0 + p
            row, tbuf, gsem, wsem = rows[p], tbufs[p], gsems[p], wsems[p]
            pltpu.make_async_copy(
                t_hbm.at[pl.ds(0, CHUNK)], row, gsem
            ).wait()

            @pl.when(t + 1 < NBLK)
            def _():
                fire_gather(t + 1, rows[1 - p], gsems[1 - p])

            @pl.when(t >= 2)
            def _():
                pltpu.make_async_copy(
                    tbuf, out_hbm.at[0, :, 0], wsem
                ).wait()

            _block_transpose(row, tbuf)
            bl = wid * NBLK + t
            pltpu.async_copy(tbuf, out_hbm.at[bl // 128, :, bl % 128], wsem)

    for p in range(2):
        pltpu.make_async_copy(
            tbufs[p], out_hbm.at[0, :, 0], wsems[p]
        ).wait()


def kernel(x, table):
    t2 = _table_transpose(table.T)
    tl = t2.reshape(V, D)
    xtf = x.T.reshape(B).astype(jnp.int32)
    o5 = _gather(xtf, tl)
    return (o5.reshape(50, 8, 128, 8, 128)
            .transpose(2, 4, 0, 1, 3)
            .reshape(x.shape[0], x.shape[1], D))


# speedup vs baseline: 5.3875x; 3.5388x over previous
import functools

import jax
import jax.numpy as jnp
from jax import lax
from jax.experimental import pallas as pl
from jax.experimental.pallas import tpu as pltpu
from jax.experimental.pallas import tpu_sc as plsc

NC, NS = 2, 16
NW = NC * NS
B = 16384 * 50
D = 64
V = 1_000_000
CHUNK = 128
PER_W = B // NW
NBLK = PER_W // CHUNK

VBLK = V // CHUNK
VTAIL = V - VBLK * CHUNK
SLOTS0 = 246

_mesh = plsc.VectorSubcoreMesh(
    core_axis_name="c", subcore_axis_name="s", num_cores=NC, num_subcores=NS
)


def _iota16():
    return lax.iota(jnp.int32, 16)


def _pair_transpose(src, dst, nil):
    iota = _iota16()

    @pl.loop(0, 16)
    def _(s):
        rot = (iota + s) % 16
        pend = []

        def flush():
            for v, d0, d1 in pend:
                plsc.store_scatter(dst, [d0, d1], v)
            pend.clear()

        for il0 in range(0, nil, 16):
            iv = iota + il0
            q0 = iv // 2
            h64 = (iv % 2) * D
            for c0 in range(0, D, 16):
                cv = rot + c0
                v = plsc.load_gather(src, [cv, iv])
                pend.append((v, q0, h64 + cv))
                if len(pend) == 8:
                    flush()
        flush()


def _block_transpose(row, tbuf):
    iota = _iota16()
    ivs = [iota + i0 for i0 in range(0, CHUNK, 16)]

    @pl.loop(0, 16)
    def _(s):
        rot = (iota + s) % 16
        pend = []

        def flush():
            for v, d0, d1, d2 in pend:
                plsc.store_scatter(tbuf, [d0, d1, d2], v)
            pend.clear()

        for c0 in range(0, D, 16):
            cv = rot + c0
            c8 = cv // 8
            cm = cv % 8
            for iv in ivs:
                v = plsc.load_gather(row, [iv, cv])
                pend.append((v, c8, cm, iv))
                if len(pend) == 8:
                    flush()
        flush()


@functools.partial(
    pl.kernel,
    out_type=jax.ShapeDtypeStruct((V // 2, 2 * D), jnp.float32),
    mesh=_mesh,
    scratch_types=[
        pltpu.VMEM((D, CHUNK), jnp.float32),
        pltpu.VMEM((D, CHUNK), jnp.float32),
        pltpu.VMEM((D, CHUNK), jnp.float32),
        pltpu.VMEM((D, CHUNK), jnp.float32),
        pltpu.VMEM((D, D), jnp.float32),
        pltpu.VMEM((D // 2, CHUNK), jnp.float32),
        pltpu.SemaphoreType.DMA,
        pltpu.SemaphoreType.DMA,
        pltpu.SemaphoreType.DMA,
        pltpu.SemaphoreType.DMA,
    ],
    compiler_params=pltpu.CompilerParams(use_tc_tiling_on_sc=True,
                                         needs_layout_passes=False),
)
def _table_transpose(tt_hbm, t2_hbm, stage0, stage1, tbuf0, tbuf1,
                     stail, ttail, rsem0, rsem1, wsem0, wsem1):
    wid = lax.axis_index("s") * NC + lax.axis_index("c")
    nb_w = jnp.where(wid < VBLK - (VBLK // NW) * NW, VBLK // NW + 1,
                     VBLK // NW)
    stages = (stage0, stage1)
    tbufs = (tbuf0, tbuf1)
    rsems = (rsem0, rsem1)
    wsems = (wsem0, wsem1)

    def blk(t):
        return wid + NW * t

    def fire_read(t, stage, rsem):
        pltpu.async_copy(tt_hbm.at[:, pl.ds(blk(t) * CHUNK, CHUNK)], stage,
                         rsem)

    fire_read(0, stage0, rsem0)

    @pl.loop(0, SLOTS0, step=2)
    def _(t0):
        for p in range(2):
            t = t0 + p
            stage, tbuf, rsem, wsem = stages[p], tbufs[p], rsems[p], wsems[p]

            @pl.when(t < nb_w)
            def _():
                pltpu.make_async_copy(
                    tt_hbm.at[:, pl.ds(0, CHUNK)], stage, rsem
                ).wait()

                @pl.when(t + 1 < nb_w)
                def _():
                    fire_read(t + 1, stages[1 - p], rsems[1 - p])

                @pl.when(t >= 2)
                def _():
                    pltpu.make_async_copy(
                        tbuf, t2_hbm.at[pl.ds(0, D), :], wsem
                    ).wait()

                _pair_transpose(stage, tbuf, CHUNK)
                pltpu.async_copy(
                    tbuf, t2_hbm.at[pl.ds(blk(t) * (CHUNK // 2), D), :], wsem)

    for p in range(2):
        @pl.when(nb_w >= 2 - p)
        def _():
            pltpu.make_async_copy(
                tbufs[p], t2_hbm.at[pl.ds(0, D), :], wsems[p]
            ).wait()

    @pl.when(wid == 0)
    def _():
        pltpu.sync_copy(tt_hbm.at[:, pl.ds(VBLK * CHUNK, VTAIL)], stail)
        _pair_transpose(stail, ttail, VTAIL)
        pltpu.sync_copy(ttail, t2_hbm.at[pl.ds(VBLK * (CHUNK // 2),
                                               VTAIL // 2), :])


@functools.partial(
    pl.kernel,
    out_type=jax.ShapeDtypeStruct((50, 8, 128, 8, 128), jnp.float32),
    mesh=_mesh,
    scratch_types=[
        pltpu.VMEM((PER_W,), jnp.int32),
        pltpu.VMEM((CHUNK, D), jnp.float32),
        pltpu.VMEM((CHUNK, D), jnp.float32),
        pltpu.VMEM((8, 8, CHUNK), jnp.float32),
        pltpu.VMEM((8, 8, CHUNK), jnp.float32),
        pltpu.SemaphoreType.DMA,
        pltpu.SemaphoreType.DMA,
        pltpu.SemaphoreType.DMA,
        pltpu.SemaphoreType.DMA,
    ],
    compiler_params=pltpu.CompilerParams(use_tc_tiling_on_sc=False,
                                         needs_layout_passes=False),
)
def _gather(x_hbm, t_hbm, out_hbm, idx_v, rows0, rows1, tbuf0, tbuf1,
            gsem0, gsem1, wsem0, wsem1):
    wid = lax.axis_index("s") * NC + lax.axis_index("c")
    pltpu.sync_copy(x_hbm.at[pl.ds(wid * PER_W, PER_W)], idx_v)
    rows = (rows0, rows1)
    tbufs = (tbuf0, tbuf1)
    gsems = (gsem0, gsem1)
    wsems = (wsem0, wsem1)

    def fire_gather(t, dst, gsem):
        pltpu.async_copy(
            t_hbm.at[idx_v.at[pl.ds(t * CHUNK, CHUNK)]], dst, gsem)

    fire_gather(0, rows0, gsem0)

    @pl.loop(0, NBLK, step=2)
    def _(t0):
        for p in range(2):
            t = t0 + p
            row, tbuf, gsem, wsem = rows[p], tbufs[p], gsems[p], wsems[p]
            pltpu.make_async_copy(
                t_hbm.at[pl.ds(0, CHUNK)], row, gsem
            ).wait()

            @pl.when(t + 1 < NBLK)
            def _():
                fire_gather(t + 1, rows[1 - p], gsems[1 - p])

            @pl.when(t >= 2)
            def _():
                pltpu.make_async_copy(
                    tbuf, out_hbm.at[0, :, 0], wsem
                ).wait()

            _block_transpose(row, tbuf)
            bl = wid * NBLK + t
            pltpu.async_copy(tbuf, out_hbm.at[bl // 128, :, bl % 128], wsem)

    for p in range(2):
        pltpu.make_async_copy(
            tbufs[p], out_hbm.at[0, :, 0], wsems[p]
        ).wait()


def kernel(x, table):
    t2 = _table_transpose(table.T)
    tl = t2.reshape(V, D)
    xtf = x.T.reshape(B).astype(jnp.int32)
    o5 = _gather(xtf, tl)
    return (o5.transpose(2, 4, 0, 1, 3)
            .reshape(x.shape[0], x.shape[1], D))
